# Initial kernel scaffold; baseline (speedup 1.0000x reference)
import functools

import jax
import jax.numpy as jnp
from jax import lax
from jax.experimental import pallas as pl
from jax.experimental.pallas import tpu as pltpu
from jax.experimental.pallas import tpu_sc as plsc

N_NODES = 10000
N_EDGES = 320000
D_FEAT = 128
HIDDEN = 16
N_CLASSES = 10

NC = 2
NS = 16
NW = NC * NS
EW = N_EDGES // NW
KC = 100
CH = EW // KC
RW = N_NODES // NS

_mesh = plsc.VectorSubcoreMesh(core_axis_name="c", subcore_axis_name="s")


@functools.partial(
    pl.kernel,
    mesh=_mesh,
    out_type=jax.ShapeDtypeStruct((NC, N_NODES, HIDDEN), jnp.float32),
    scratch_types=[
        pltpu.VMEM((CH, KC), jnp.int32),
        pltpu.VMEM((KC, HIDDEN), jnp.float32),
        pltpu.VMEM_SHARED((N_NODES, HIDDEN), jnp.float32),
    ],
)
def _deg_kernel(col_hbm, ones_hbm, zeros_hbm, out_hbm, col_v, ones_v, acc):
    cid = lax.axis_index("c")
    sid = lax.axis_index("s")
    wid = sid * NC + cid
    pltpu.sync_copy(col_hbm.at[wid], col_v)
    pltpu.sync_copy(ones_hbm, ones_v)
    pltpu.sync_copy(zeros_hbm.at[pl.ds(sid * RW, RW)], acc.at[pl.ds(sid * RW, RW)])
    plsc.subcore_barrier()

    def body(j, carry):
        pltpu.sync_copy(ones_v, acc.at[col_v.at[j]], add=True)
        return carry

    lax.fori_loop(0, CH, body, 0)
    plsc.subcore_barrier()
    pltpu.sync_copy(acc.at[pl.ds(sid * RW, RW)],
                    out_hbm.at[cid, pl.ds(sid * RW, RW)])


@functools.partial(
    pl.kernel,
    mesh=_mesh,
    out_type=jax.ShapeDtypeStruct((NC, N_NODES, HIDDEN), jnp.float32),
    scratch_types=[
        pltpu.VMEM((CH, KC), jnp.int32),
        pltpu.VMEM((CH, KC), jnp.int32),
        pltpu.VMEM((KC, HIDDEN), jnp.float32),
        pltpu.VMEM_SHARED((N_NODES, HIDDEN), jnp.float32),
        pltpu.SemaphoreType.DMA,
    ],
)
def _scatter_kernel(hs_hbm, row_hbm, col_hbm, zeros_hbm, out_hbm,
                    row_v, col_v, gat_v, acc, sem):
    cid = lax.axis_index("c")
    sid = lax.axis_index("s")
    wid = sid * NC + cid
    pltpu.sync_copy(row_hbm.at[wid], row_v)
    pltpu.sync_copy(col_hbm.at[wid], col_v)
    pltpu.sync_copy(zeros_hbm.at[pl.ds(sid * RW, RW)], acc.at[pl.ds(sid * RW, RW)])
    plsc.subcore_barrier()

    def body(j, carry):
        pltpu.async_copy(hs_hbm.at[row_v.at[j]], gat_v, sem).wait()
        pltpu.sync_copy(gat_v, acc.at[col_v.at[j]], add=True)
        return carry

    lax.fori_loop(0, CH, body, 0)
    plsc.subcore_barrier()
    pltpu.sync_copy(acc.at[pl.ds(sid * RW, RW)],
                    out_hbm.at[cid, pl.ds(sid * RW, RW)])


def _tc1_body(x_ref, w1_ref, degp_ref, hs_ref, dis_ref):
    deg = degp_ref[0] + degp_ref[1] + 1.0
    dis = lax.rsqrt(deg)
    h = jnp.dot(x_ref[...], w1_ref[...], preferred_element_type=jnp.float32)
    dis_ref[...] = dis
    hs_ref[...] = h * dis


def _tc2_body(p_ref, hs_ref, dis_ref, b1_ref, w2_ref, h2s_ref):
    dis = dis_ref[...]
    h1 = jnp.maximum(dis * (p_ref[0] + p_ref[1] + hs_ref[...]) + b1_ref[...], 0.0)
    h2s_ref[...] = jnp.dot(h1, w2_ref[...],
                           preferred_element_type=jnp.float32) * dis


def _tc3_body(q_ref, h2s_ref, dis_ref, b2_ref, out_ref):
    out_ref[...] = dis_ref[...] * (q_ref[0] + q_ref[1] + h2s_ref[...]) + b2_ref[...]


_SDS = jax.ShapeDtypeStruct


def kernel(x, edge_index, W1, b1, W2, b2):
    ei = edge_index.astype(jnp.int32)
    row3 = ei[0].reshape(NW, CH, KC)
    col3 = ei[1].reshape(NW, CH, KC)
    ones_h = jnp.ones((KC, HIDDEN), jnp.float32)
    zeros_h = jnp.zeros((N_NODES, HIDDEN), jnp.float32)
    W2p = jnp.pad(W2, ((0, 0), (0, HIDDEN - N_CLASSES)))
    b1r = b1.reshape(1, HIDDEN)
    b2p = jnp.pad(b2, (0, HIDDEN - N_CLASSES)).reshape(1, HIDDEN)

    degp = _deg_kernel(col3, ones_h, zeros_h)

    hs, dis = pl.pallas_call(
        _tc1_body,
        out_shape=(_SDS((N_NODES, HIDDEN), jnp.float32),
                   _SDS((N_NODES, HIDDEN), jnp.float32)),
    )(x, W1, degp)

    p = _scatter_kernel(hs, row3, col3, zeros_h)

    h2s = pl.pallas_call(
        _tc2_body,
        out_shape=_SDS((N_NODES, HIDDEN), jnp.float32),
    )(p, hs, dis, b1r, W2p)

    q = _scatter_kernel(h2s, row3, col3, zeros_h)

    out16 = pl.pallas_call(
        _tc3_body,
        out_shape=_SDS((N_NODES, HIDDEN), jnp.float32),
    )(q, h2s, dis, b2p)

    return out16[:, :N_CLASSES]


# speedup vs baseline: 32.0848x; 32.0848x over previous
import functools

import jax
import jax.numpy as jnp
from jax import lax
from jax.experimental import pallas as pl
from jax.experimental.pallas import tpu as pltpu
from jax.experimental.pallas import tpu_sc as plsc

N_NODES = 10000
N_EDGES = 320000
D_FEAT = 128
HIDDEN = 16
N_CLASSES = 10

NC = 2
NS = 16
NW = NC * NS
EW = N_EDGES // NW
KC = 100
CH = EW // KC
NP = 10240
RW = NP // NS

_mesh = plsc.VectorSubcoreMesh(core_axis_name="c", subcore_axis_name="s")


@functools.partial(
    pl.kernel,
    mesh=_mesh,
    compiler_params=pltpu.CompilerParams(use_tc_tiling_on_sc=False),
    out_type=jax.ShapeDtypeStruct((NC, NP, HIDDEN), jnp.float32),
    scratch_types=[
        pltpu.VMEM((CH, KC), jnp.int32),
        pltpu.VMEM((KC, HIDDEN), jnp.float32),
        pltpu.VMEM_SHARED((NP, HIDDEN), jnp.float32),
    ],
)
def _deg_kernel(col_hbm, ones_hbm, zeros_hbm, out_hbm, col_v, ones_v, acc):
    cid = lax.axis_index("c")
    sid = lax.axis_index("s")
    wid = sid * NC + cid
    pltpu.sync_copy(col_hbm.at[wid], col_v)
    pltpu.sync_copy(ones_hbm, ones_v)
    pltpu.sync_copy(zeros_hbm.at[pl.ds(sid * RW, RW)], acc.at[pl.ds(sid * RW, RW)])
    plsc.subcore_barrier()

    def body(j, carry):
        pltpu.sync_copy(ones_v, acc.at[col_v.at[j]], add=True)
        return carry

    lax.fori_loop(0, CH, body, 0)
    plsc.subcore_barrier()
    pltpu.sync_copy(acc.at[pl.ds(sid * RW, RW)],
                    out_hbm.at[cid, pl.ds(sid * RW, RW)])


@functools.partial(
    pl.kernel,
    mesh=_mesh,
    compiler_params=pltpu.CompilerParams(use_tc_tiling_on_sc=False),
    out_type=jax.ShapeDtypeStruct((NC, NP, HIDDEN), jnp.float32),
    scratch_types=[
        pltpu.VMEM((CH, KC), jnp.int32),
        pltpu.VMEM((CH, KC), jnp.int32),
        pltpu.VMEM((KC, HIDDEN), jnp.float32),
        pltpu.VMEM_SHARED((NP, HIDDEN), jnp.float32),
        pltpu.SemaphoreType.DMA,
    ],
)
def _scatter_kernel(hs_hbm, row_hbm, col_hbm, zeros_hbm, out_hbm,
                    row_v, col_v, gat_v, acc, sem):
    cid = lax.axis_index("c")
    sid = lax.axis_index("s")
    wid = sid * NC + cid
    pltpu.sync_copy(row_hbm.at[wid], row_v)
    pltpu.sync_copy(col_hbm.at[wid], col_v)
    pltpu.sync_copy(zeros_hbm.at[pl.ds(sid * RW, RW)], acc.at[pl.ds(sid * RW, RW)])
    plsc.subcore_barrier()

    def body(j, carry):
        pltpu.async_copy(hs_hbm.at[row_v.at[j]], gat_v, sem).wait()
        pltpu.sync_copy(gat_v, acc.at[col_v.at[j]], add=True)
        return carry

    lax.fori_loop(0, CH, body, 0)
    plsc.subcore_barrier()
    pltpu.sync_copy(acc.at[pl.ds(sid * RW, RW)],
                    out_hbm.at[cid, pl.ds(sid * RW, RW)])


def _tc1_body(x_ref, w1_ref, degp_ref, hs_ref, dis_ref):
    deg = degp_ref[0, :N_NODES] + degp_ref[1, :N_NODES] + 1.0
    dis = lax.rsqrt(deg)
    h = jnp.dot(x_ref[...], w1_ref[...], preferred_element_type=jnp.float32)
    dis_ref[...] = dis
    hs_ref[...] = h * dis


def _tc2_body(p_ref, hs_ref, dis_ref, b1_ref, w2_ref, h2s_ref):
    dis = dis_ref[...]
    h1 = jnp.maximum(dis * (p_ref[0, :N_NODES] + p_ref[1, :N_NODES] + hs_ref[...]) + b1_ref[...], 0.0)
    h2s_ref[...] = jnp.dot(h1, w2_ref[...],
                           preferred_element_type=jnp.float32) * dis


def _tc3_body(q_ref, h2s_ref, dis_ref, b2_ref, out_ref):
    out_ref[...] = dis_ref[...] * (q_ref[0, :N_NODES] + q_ref[1, :N_NODES] + h2s_ref[...]) + b2_ref[...]


_SDS = jax.ShapeDtypeStruct


def kernel(x, edge_index, W1, b1, W2, b2):
    ei = edge_index.astype(jnp.int32)
    row3 = ei[0].reshape(NW, CH, KC)
    col3 = ei[1].reshape(NW, CH, KC)
    ones_h = jnp.ones((KC, HIDDEN), jnp.float32)
    zeros_h = jnp.zeros((NP, HIDDEN), jnp.float32)
    W2p = jnp.pad(W2, ((0, 0), (0, HIDDEN - N_CLASSES)))
    b1r = b1.reshape(1, HIDDEN)
    b2p = jnp.pad(b2, (0, HIDDEN - N_CLASSES)).reshape(1, HIDDEN)

    degp = _deg_kernel(col3, ones_h, zeros_h)

    hs, dis = pl.pallas_call(
        _tc1_body,
        out_shape=(_SDS((N_NODES, HIDDEN), jnp.float32),
                   _SDS((N_NODES, HIDDEN), jnp.float32)),
    )(x, W1, degp)

    p = _scatter_kernel(hs, row3, col3, zeros_h)

    h2s = pl.pallas_call(
        _tc2_body,
        out_shape=_SDS((N_NODES, HIDDEN), jnp.float32),
    )(p, hs, dis, b1r, W2p)

    q = _scatter_kernel(h2s, row3, col3, zeros_h)

    out16 = pl.pallas_call(
        _tc3_body,
        out_shape=_SDS((N_NODES, HIDDEN), jnp.float32),
    )(q, h2s, dis, b2p)

    return out16[:, :N_CLASSES]


# speedup vs baseline: 59.7753x; 1.8630x over previous
import functools

import jax
import jax.numpy as jnp
from jax import lax
from jax.experimental import pallas as pl
from jax.experimental.pallas import tpu as pltpu
from jax.experimental.pallas import tpu_sc as plsc

N_NODES = 10000
N_EDGES = 320000
D_FEAT = 128
HIDDEN = 16
N_CLASSES = 10

NC = 2
NS = 16
NW = NC * NS
EW = N_EDGES // NW
KC = 125
CH = EW // KC
NBUF = 8
NG = CH // NBUF
NP = 10240
RW = NP // NS

_mesh = plsc.VectorSubcoreMesh(core_axis_name="c", subcore_axis_name="s")
_params = pltpu.CompilerParams(use_tc_tiling_on_sc=False)


@functools.partial(
    pl.kernel,
    mesh=_mesh,
    compiler_params=_params,
    out_type=jax.ShapeDtypeStruct((NC, NP, HIDDEN), jnp.float32),
    scratch_types=[
        pltpu.VMEM((CH, KC), jnp.int32),
        pltpu.VMEM((KC, HIDDEN), jnp.float32),
        pltpu.VMEM_SHARED((NP, HIDDEN), jnp.float32),
        pltpu.SemaphoreType.DMA((NBUF,)),
    ],
)
def _deg_kernel(col_hbm, ones_hbm, zeros_hbm, out_hbm, col_v, ones_v, acc, sems):
    cid = lax.axis_index("c")
    sid = lax.axis_index("s")
    wid = sid * NC + cid
    pltpu.sync_copy(col_hbm.at[wid], col_v)
    pltpu.sync_copy(ones_hbm, ones_v)
    pltpu.sync_copy(zeros_hbm.at[pl.ds(sid * RW, RW)], acc.at[pl.ds(sid * RW, RW)])
    plsc.subcore_barrier()

    def group(g, carry):
        base = g * NBUF
        for b in range(NBUF):
            pltpu.async_copy(ones_v, acc.at[col_v.at[base + b]], sems.at[b],
                             add=True)
        for b in range(NBUF):
            pltpu.make_async_copy(ones_v, acc.at[col_v.at[base + b]],
                                  sems.at[b]).wait()
        return carry

    lax.fori_loop(0, NG, group, 0)
    plsc.subcore_barrier()
    pltpu.sync_copy(acc.at[pl.ds(sid * RW, RW)],
                    out_hbm.at[cid, pl.ds(sid * RW, RW)])


@functools.partial(
    pl.kernel,
    mesh=_mesh,
    compiler_params=_params,
    out_type=jax.ShapeDtypeStruct((NC, NP, HIDDEN), jnp.float32),
    scratch_types=[
        pltpu.VMEM((CH, KC), jnp.int32),
        pltpu.VMEM((CH, KC), jnp.int32),
        pltpu.VMEM((NBUF, KC, HIDDEN), jnp.float32),
        pltpu.VMEM_SHARED((NP, HIDDEN), jnp.float32),
        pltpu.SemaphoreType.DMA((NBUF,)),
        pltpu.SemaphoreType.DMA((NBUF,)),
    ],
)
def _scatter_kernel(hs_hbm, row_hbm, col_hbm, zeros_hbm, out_hbm,
                    row_v, col_v, gat_v, acc, semg, sems):
    cid = lax.axis_index("c")
    sid = lax.axis_index("s")
    wid = sid * NC + cid
    pltpu.sync_copy(row_hbm.at[wid], row_v)
    pltpu.sync_copy(col_hbm.at[wid], col_v)
    pltpu.sync_copy(zeros_hbm.at[pl.ds(sid * RW, RW)], acc.at[pl.ds(sid * RW, RW)])
    plsc.subcore_barrier()

    for b in range(NBUF):
        pltpu.async_copy(hs_hbm.at[row_v.at[b]], gat_v.at[b], semg.at[b])

    def group(g, carry):
        base = g * NBUF
        for b in range(NBUF):
            jj = base + b
            pltpu.make_async_copy(hs_hbm.at[row_v.at[jj]], gat_v.at[b],
                                  semg.at[b]).wait()
            pltpu.async_copy(gat_v.at[b], acc.at[col_v.at[jj]], sems.at[b],
                             add=True)
        for b in range(NBUF):
            jj = base + b
            pltpu.make_async_copy(gat_v.at[b], acc.at[col_v.at[jj]],
                                  sems.at[b]).wait()

            @pl.when(g < NG - 1)
            def _():
                pltpu.async_copy(hs_hbm.at[row_v.at[jj + NBUF]], gat_v.at[b],
                                 semg.at[b])

        return carry

    lax.fori_loop(0, NG, group, 0)
    plsc.subcore_barrier()
    pltpu.sync_copy(acc.at[pl.ds(sid * RW, RW)],
                    out_hbm.at[cid, pl.ds(sid * RW, RW)])


def _tc1_body(x_ref, w1_ref, degp_ref, hs_ref, dis_ref):
    deg = degp_ref[0, :N_NODES] + degp_ref[1, :N_NODES] + 1.0
    dis = lax.rsqrt(deg)
    h = jnp.dot(x_ref[...], w1_ref[...], preferred_element_type=jnp.float32)
    dis_ref[...] = dis
    hs_ref[...] = h * dis


def _tc2_body(p_ref, hs_ref, dis_ref, b1_ref, w2_ref, h2s_ref):
    dis = dis_ref[...]
    h1 = jnp.maximum(
        dis * (p_ref[0, :N_NODES] + p_ref[1, :N_NODES] + hs_ref[...])
        + b1_ref[...], 0.0)
    h2s_ref[...] = jnp.dot(h1, w2_ref[...],
                           preferred_element_type=jnp.float32) * dis


def _tc3_body(q_ref, h2s_ref, dis_ref, b2_ref, out_ref):
    out_ref[...] = dis_ref[...] * (
        q_ref[0, :N_NODES] + q_ref[1, :N_NODES] + h2s_ref[...]) + b2_ref[...]


_SDS = jax.ShapeDtypeStruct


def kernel(x, edge_index, W1, b1, W2, b2):
    ei = edge_index.astype(jnp.int32)
    row3 = ei[0].reshape(NW, CH, KC)
    col3 = ei[1].reshape(NW, CH, KC)
    ones_h = jnp.ones((KC, HIDDEN), jnp.float32)
    zeros_h = jnp.zeros((NP, HIDDEN), jnp.float32)
    W2p = jnp.pad(W2, ((0, 0), (0, HIDDEN - N_CLASSES)))
    b1r = b1.reshape(1, HIDDEN)
    b2p = jnp.pad(b2, (0, HIDDEN - N_CLASSES)).reshape(1, HIDDEN)

    degp = _deg_kernel(col3, ones_h, zeros_h)

    hs, dis = pl.pallas_call(
        _tc1_body,
        out_shape=(_SDS((N_NODES, HIDDEN), jnp.float32),
                   _SDS((N_NODES, HIDDEN), jnp.float32)),
    )(x, W1, degp)

    p = _scatter_kernel(hs, row3, col3, zeros_h)

    h2s = pl.pallas_call(
        _tc2_body,
        out_shape=_SDS((N_NODES, HIDDEN), jnp.float32),
    )(p, hs, dis, b1r, W2p)

    q = _scatter_kernel(h2s, row3, col3, zeros_h)

    out16 = pl.pallas_call(
        _tc3_body,
        out_shape=_SDS((N_NODES, HIDDEN), jnp.float32),
    )(q, h2s, dis, b2p)

    return out16[:, :N_CLASSES]


# speedup vs baseline: 70.1472x; 1.1735x over previous
import functools

import jax
import jax.numpy as jnp
from jax import lax
from jax.experimental import pallas as pl
from jax.experimental.pallas import tpu as pltpu
from jax.experimental.pallas import tpu_sc as plsc

N_NODES = 10000
N_EDGES = 320000
D_FEAT = 128
HIDDEN = 16
N_CLASSES = 10

NC = 2
NS = 16
NW = NC * NS
EW = N_EDGES // NW
KC = 125
CH = EW // KC
NBUF = 8
NG = CH // NBUF
NP = 10240
RW = NP // NS

_mesh = plsc.VectorSubcoreMesh(core_axis_name="c", subcore_axis_name="s")
_params = pltpu.CompilerParams(use_tc_tiling_on_sc=False)


@functools.partial(
    pl.kernel,
    mesh=_mesh,
    compiler_params=_params,
    out_type=jax.ShapeDtypeStruct((NC, NP, 128), jnp.float32),
    scratch_types=[
        pltpu.VMEM((CH, KC), jnp.int32),
        pltpu.VMEM((KC, HIDDEN), jnp.float32),
        pltpu.VMEM_SHARED((NP, HIDDEN), jnp.float32),
        pltpu.SemaphoreType.DMA((NBUF,)),
    ],
)
def _deg_kernel(col_hbm, ones_hbm, zeros_hbm, out_hbm, col_v, ones_v, acc, sems):
    cid = lax.axis_index("c")
    sid = lax.axis_index("s")
    wid = sid * NC + cid
    pltpu.sync_copy(col_hbm.at[wid], col_v)
    pltpu.sync_copy(ones_hbm, ones_v)
    pltpu.sync_copy(zeros_hbm.at[pl.ds(sid * RW, RW)], acc.at[pl.ds(sid * RW, RW)])
    plsc.subcore_barrier()

    def group(g, carry):
        base = g * NBUF
        for b in range(NBUF):
            pltpu.async_copy(ones_v, acc.at[col_v.at[base + b]], sems.at[b],
                             add=True)
        for b in range(NBUF):
            pltpu.make_async_copy(ones_v, acc.at[col_v.at[base + b]],
                                  sems.at[b]).wait()
        return carry

    lax.fori_loop(0, NG, group, 0)
    plsc.subcore_barrier()
    pltpu.sync_copy(acc.at[pl.ds(sid * RW, RW)],
                    out_hbm.at[cid, pl.ds(sid * RW, RW), pl.ds(0, HIDDEN)])


@functools.partial(
    pl.kernel,
    mesh=_mesh,
    compiler_params=_params,
    out_type=jax.ShapeDtypeStruct((NC, NP, 128), jnp.float32),
    scratch_types=[
        pltpu.VMEM((CH, KC), jnp.int32),
        pltpu.VMEM((CH, KC), jnp.int32),
        pltpu.VMEM((NBUF, KC, HIDDEN), jnp.float32),
        pltpu.VMEM_SHARED((NP, HIDDEN), jnp.float32),
        pltpu.SemaphoreType.DMA((NBUF,)),
        pltpu.SemaphoreType.DMA((NBUF,)),
    ],
)
def _scatter_kernel(hs_hbm, row_hbm, col_hbm, zeros_hbm, out_hbm,
                    row_v, col_v, gat_v, acc, semg, sems):
    cid = lax.axis_index("c")
    sid = lax.axis_index("s")
    wid = sid * NC + cid
    pltpu.sync_copy(row_hbm.at[wid], row_v)
    pltpu.sync_copy(col_hbm.at[wid], col_v)
    pltpu.sync_copy(zeros_hbm.at[pl.ds(sid * RW, RW)], acc.at[pl.ds(sid * RW, RW)])
    plsc.subcore_barrier()

    for b in range(NBUF):
        pltpu.async_copy(hs_hbm.at[row_v.at[b]], gat_v.at[b], semg.at[b])

    def group(g, carry):
        base = g * NBUF
        for b in range(NBUF):
            jj = base + b
            pltpu.make_async_copy(hs_hbm.at[row_v.at[jj]], gat_v.at[b],
                                  semg.at[b]).wait()
            pltpu.async_copy(gat_v.at[b], acc.at[col_v.at[jj]], sems.at[b],
                             add=True)
        for b in range(NBUF):
            jj = base + b
            pltpu.make_async_copy(gat_v.at[b], acc.at[col_v.at[jj]],
                                  sems.at[b]).wait()

            @pl.when(g < NG - 1)
            def _():
                pltpu.async_copy(hs_hbm.at[row_v.at[jj + NBUF]], gat_v.at[b],
                                 semg.at[b])

        return carry

    lax.fori_loop(0, NG, group, 0)
    plsc.subcore_barrier()
    pltpu.sync_copy(acc.at[pl.ds(sid * RW, RW)],
                    out_hbm.at[cid, pl.ds(sid * RW, RW), pl.ds(0, HIDDEN)])


def _tc0_body(ei_ref, row_ref, col_ref):
    row_ref[...] = ei_ref[0, :]
    col_ref[...] = ei_ref[1, :]


def _tc1a_body(x_ref, w1_ref, h_ref):
    h_ref[...] = jnp.dot(x_ref[...], w1_ref[...],
                         preferred_element_type=jnp.float32)


def _tc1b_body(h_ref, degp_ref, hs_ref, dis_ref):
    deg = degp_ref[0, :N_NODES, :HIDDEN] + degp_ref[1, :N_NODES, :HIDDEN] + 1.0
    dis = lax.rsqrt(deg)
    dis_ref[...] = dis
    hs_ref[...] = h_ref[...] * dis


def _tc2_body(p_ref, hs_ref, dis_ref, b1_ref, w2_ref, h2s_ref):
    dis = dis_ref[...]
    h1 = jnp.maximum(
        dis * (p_ref[0, :N_NODES, :HIDDEN] + p_ref[1, :N_NODES, :HIDDEN] + hs_ref[...])
        + b1_ref[...], 0.0)
    h2s_ref[...] = jnp.dot(h1, w2_ref[...],
                           preferred_element_type=jnp.float32) * dis


def _tc3_body(q_ref, h2s_ref, dis_ref, b2_ref, out_ref):
    out_ref[...] = dis_ref[...] * (
        q_ref[0, :N_NODES, :HIDDEN] + q_ref[1, :N_NODES, :HIDDEN] + h2s_ref[...]) + b2_ref[...]


_SDS = jax.ShapeDtypeStruct


def kernel(x, edge_index, W1, b1, W2, b2):
    ei = edge_index.astype(jnp.int32)
    row1, col1 = pl.pallas_call(
        _tc0_body,
        out_shape=(_SDS((N_EDGES,), jnp.int32), _SDS((N_EDGES,), jnp.int32)),
    )(ei)
    row3 = row1.reshape(NW, CH, KC)
    col3 = col1.reshape(NW, CH, KC)
    ones_h = jnp.ones((KC, HIDDEN), jnp.float32)
    zeros_h = jnp.zeros((NP, HIDDEN), jnp.float32)
    W2p = jnp.pad(W2, ((0, 0), (0, HIDDEN - N_CLASSES)))
    b1r = b1.reshape(1, HIDDEN)
    b2p = jnp.pad(b2, (0, HIDDEN - N_CLASSES)).reshape(1, HIDDEN)

    degp = _deg_kernel(col3, ones_h, zeros_h)

    h = pl.pallas_call(
        _tc1a_body,
        out_shape=_SDS((N_NODES, HIDDEN), jnp.float32),
    )(x, W1)

    hs, dis = pl.pallas_call(
        _tc1b_body,
        out_shape=(_SDS((N_NODES, HIDDEN), jnp.float32),
                   _SDS((N_NODES, HIDDEN), jnp.float32)),
    )(h, degp)

    p = _scatter_kernel(hs, row3, col3, zeros_h)

    h2s = pl.pallas_call(
        _tc2_body,
        out_shape=_SDS((N_NODES, HIDDEN), jnp.float32),
    )(p, hs, dis, b1r, W2p)

    q = _scatter_kernel(h2s, row3, col3, zeros_h)

    out16 = pl.pallas_call(
        _tc3_body,
        out_shape=_SDS((N_NODES, HIDDEN), jnp.float32),
    )(q, h2s, dis, b2p)

    return out16[:, :N_CLASSES]
